# baseline (speedup 1.0000x reference)
import functools

import jax
import jax.numpy as jnp
from jax import lax
from jax.experimental import pallas as pl
from jax.experimental.pallas import tpu as pltpu
from jax.experimental.pallas import tpu_sc as plsc

_L = 16
_NUM_FACTORS = 32
_TR = 128


def _build_transpose(num_rows, num_workers, nc):
    n_full = num_rows // _TR
    tail = num_rows - n_full * _TR
    n_main = (n_full // (2 * num_workers)) * 2
    n_left = n_full - n_main * num_workers
    mesh = plsc.VectorSubcoreMesh(core_axis_name="c", subcore_axis_name="s")

    out_type = [
        jax.ShapeDtypeStruct((num_rows, _NUM_FACTORS), jnp.float32),
        jax.ShapeDtypeStruct((num_rows, _NUM_FACTORS), jnp.float32),
    ]

    @functools.partial(
        pl.kernel,
        out_type=out_type,
        mesh=mesh,
        compiler_params=pltpu.CompilerParams(needs_layout_passes=False),
        scratch_types=[
            pltpu.VMEM((2, _NUM_FACTORS, _TR), jnp.float32),
            pltpu.VMEM((2, _TR, _NUM_FACTORS), jnp.float32),
            pltpu.SemaphoreType.DMA,
            pltpu.SemaphoreType.DMA,
            pltpu.SemaphoreType.DMA,
            pltpu.SemaphoreType.DMA,
        ],
    )
    def tr_kernel(uft_hbm, pft_hbm, tail_u_hbm, tail_p_hbm, out_u, out_p,
                  blk_v, rows_v, sem_r0, sem_r1, sem_w0, sem_w1):
        wid = lax.axis_index("s") * nc + lax.axis_index("c")
        lanes = lax.iota(jnp.int32, _L)
        sem_r = (sem_r0, sem_r1)
        sem_w = (sem_w0, sem_w1)

        def fetch(src, blk, slot):
            pltpu.async_copy(
                src.at[:, pl.ds(blk * _TR, _TR)], blk_v.at[slot], sem_r[slot])

        def drain_read(src, slot):
            pltpu.make_async_copy(
                src.at[:, pl.ds(0, _TR)], blk_v.at[slot], sem_r[slot]).wait()

        def drain_write(dst, slot):
            pltpu.make_async_copy(
                rows_v.at[slot], dst.at[pl.ds(0, _TR), :], sem_w[slot]).wait()

        def assemble(slot, n_rows=_TR):
            slot_idx = jnp.full((_L,), slot, jnp.int32)

            def row_body(rr, _):
                rv = jnp.full((_L,), rr, jnp.int32)
                lo = plsc.load_gather(blk_v, [slot_idx, lanes, rv])
                hi = plsc.load_gather(blk_v, [slot_idx, lanes + _L, rv])
                rows_v[slot, rr, pl.ds(0, _L)] = lo
                rows_v[slot, rr, pl.ds(_L, _L)] = hi
                return _

            lax.fori_loop(0, n_rows, row_body, None)

        def one_table(src, dst, first):
            fetch(src, wid, 0)

            def step(j, _):
                b0 = wid + (2 * j) * num_workers
                b1 = wid + (2 * j + 1) * num_workers
                fetch(src, b1, 1)
                drain_read(src, 0)

                @pl.when(jnp.logical_or(j > 0, jnp.logical_not(first)))
                def _():
                    drain_write(dst, 0)

                assemble(0)
                pltpu.async_copy(
                    rows_v.at[0], dst.at[pl.ds(b0 * _TR, _TR), :], sem_w0)

                @pl.when(j + 1 < n_main // 2)
                def _():
                    fetch(src, b1 + num_workers, 0)

                drain_read(src, 1)

                @pl.when(jnp.logical_or(j > 0, jnp.logical_not(first)))
                def _():
                    drain_write(dst, 1)

                assemble(1)
                pltpu.async_copy(
                    rows_v.at[1], dst.at[pl.ds(b1 * _TR, _TR), :], sem_w1)
                return _

            lax.fori_loop(0, n_main // 2, step, None)

        one_table(uft_hbm, out_u, True)
        one_table(pft_hbm, out_p, False)
        drain_write(out_p, 0)
        drain_write(out_p, 1)

        if n_left:
            base_blk = n_main * num_workers

            @pl.when(wid < 2 * n_left)
            def _():
                which = wid // 2
                is_p = wid % 2
                blk = base_blk + which

                def do(src, dst):
                    fetch(src, blk, 0)
                    drain_read(src, 0)
                    assemble(0)
                    pltpu.sync_copy(
                        rows_v.at[0], dst.at[pl.ds(blk * _TR, _TR), :])

                @pl.when(is_p == 0)
                def _():
                    do(uft_hbm, out_u)

                @pl.when(is_p == 1)
                def _():
                    do(pft_hbm, out_p)

        if tail:
            r0 = n_full * _TR

            def tail_table(src, dst, owner):
                @pl.when(wid == owner)
                def _():
                    pltpu.sync_copy(src, blk_v.at[1])
                    assemble(1)
                    pltpu.sync_copy(
                        rows_v.at[1, pl.ds(0, tail), :],
                        dst.at[pl.ds(r0, tail), :])

            tail_table(tail_u_hbm, out_u, num_workers - 2)
            tail_table(tail_p_hbm, out_p, num_workers - 1)

    return tr_kernel


def _build_gather(batch, num_workers, nc):
    b_per_w = batch // num_workers
    n_groups = b_per_w // _L
    mesh = plsc.VectorSubcoreMesh(core_axis_name="c", subcore_axis_name="s")

    @functools.partial(
        pl.kernel,
        out_type=jax.ShapeDtypeStruct((batch,), jnp.float32),
        mesh=mesh,
        compiler_params=pltpu.CompilerParams(
            needs_layout_passes=False, use_tc_tiling_on_sc=False),
        scratch_types=[
            pltpu.VMEM((b_per_w,), jnp.int32),
            pltpu.VMEM((b_per_w,), jnp.int32),
            pltpu.VMEM((b_per_w, _NUM_FACTORS), jnp.float32),
            pltpu.VMEM((b_per_w, _NUM_FACTORS), jnp.float32),
            pltpu.VMEM((b_per_w,), jnp.float32),
            pltpu.VMEM((b_per_w,), jnp.float32),
            pltpu.VMEM((_L,), jnp.float32),
            pltpu.VMEM((b_per_w,), jnp.float32),
            pltpu.SemaphoreType.DMA,
            pltpu.SemaphoreType.DMA,
            pltpu.SemaphoreType.DMA,
            pltpu.SemaphoreType.DMA,
        ],
    )
    def mf_kernel(uidx_hbm, pidx_hbm, uf_hbm, pf_hbm, ub_hbm, pb_hbm, g_hbm,
                  out_hbm, uidx_v, pidx_v, urow_v, prow_v, ub_v, pb_v, g_v,
                  out_v, sem_u, sem_p, sem_ub, sem_pb):
        wid = lax.axis_index("s") * nc + lax.axis_index("c")
        base = wid * b_per_w

        pltpu.sync_copy(uidx_hbm.at[pl.ds(base, b_per_w)], uidx_v)
        pltpu.sync_copy(pidx_hbm.at[pl.ds(base, b_per_w)], pidx_v)
        pltpu.sync_copy(g_hbm, g_v)

        cp_u = pltpu.async_copy(uf_hbm.at[uidx_v], urow_v, sem_u)
        cp_p = pltpu.async_copy(pf_hbm.at[pidx_v], prow_v, sem_p)
        cp_ub = pltpu.async_copy(ub_hbm.at[uidx_v], ub_v, sem_ub)
        cp_pb = pltpu.async_copy(pb_hbm.at[pidx_v], pb_v, sem_pb)
        cp_u.wait()
        cp_p.wait()
        cp_ub.wait()
        cp_pb.wait()

        lanes = lax.iota(jnp.int32, _L)
        gvec = g_v[...]

        def group_body(g, _):
            off = pl.multiple_of(g * _L, _L)
            rows = off + lanes
            acc = gvec
            for d in range(_NUM_FACTORS):
                cols = jnp.full((_L,), d, jnp.int32)
                u = plsc.load_gather(urow_v, [rows, cols])
                p = plsc.load_gather(prow_v, [rows, cols])
                acc = acc + u * p
            out_v[pl.ds(off, _L)] = acc + ub_v[pl.ds(off, _L)] + pb_v[pl.ds(off, _L)]
            return _

        lax.fori_loop(0, n_groups, group_body, None)
        pltpu.sync_copy(out_v, out_hbm.at[pl.ds(base, b_per_w)])

    return mf_kernel


def kernel(user_indices, post_indices, user_factors, post_factors,
           user_intercepts, post_intercepts, global_intercept):
    info = plsc.get_sparse_core_info()
    nc, ns = info.num_cores, info.num_subcores
    batch = user_indices.shape[0]
    num_rows = user_factors.shape[0]
    tr = _build_transpose(num_rows, nc * ns, nc)
    tail_start = (num_rows // _TR) * _TR
    pad_n = tail_start + _TR - num_rows

    def tail_of(table):
        return jnp.pad(table.T[:, tail_start:], ((0, 0), (0, pad_n)))

    uf_rm, pf_rm = tr(user_factors.T, post_factors.T,
                      tail_of(user_factors), tail_of(post_factors))
    call = _build_gather(batch, nc * ns, nc)
    return call(
        user_indices.astype(jnp.int32),
        post_indices.astype(jnp.int32),
        uf_rm,
        pf_rm,
        user_intercepts.reshape(-1),
        post_intercepts.reshape(-1),
        jnp.broadcast_to(global_intercept.astype(jnp.float32), (_L,)),
    )


# speedup vs baseline: 2.2381x; 2.2381x over previous
import functools

import jax
import jax.numpy as jnp
from jax import lax
from jax.experimental import pallas as pl
from jax.experimental.pallas import tpu as pltpu
from jax.experimental.pallas import tpu_sc as plsc

_L = 16
_NUM_FACTORS = 32


def _build_call(batch, num_workers, nc):
    b_per_w = batch // num_workers
    n_groups = b_per_w // _L
    mesh = plsc.VectorSubcoreMesh(core_axis_name="c", subcore_axis_name="s")

    @functools.partial(
        pl.kernel,
        out_type=jax.ShapeDtypeStruct((batch,), jnp.float32),
        mesh=mesh,
        compiler_params=pltpu.CompilerParams(
            needs_layout_passes=False, use_tc_tiling_on_sc=False),
        scratch_types=[
            pltpu.VMEM((b_per_w,), jnp.int32),
            pltpu.VMEM((b_per_w,), jnp.int32),
            pltpu.VMEM((b_per_w, _NUM_FACTORS), jnp.float32),
            pltpu.VMEM((b_per_w, _NUM_FACTORS), jnp.float32),
            pltpu.VMEM((b_per_w,), jnp.float32),
            pltpu.VMEM((b_per_w,), jnp.float32),
            pltpu.VMEM((_L,), jnp.float32),
            pltpu.VMEM((b_per_w,), jnp.float32),
            pltpu.SemaphoreType.DMA,
            pltpu.SemaphoreType.DMA,
            pltpu.SemaphoreType.DMA,
            pltpu.SemaphoreType.DMA,
        ],
    )
    def mf_kernel(uidx_hbm, pidx_hbm, uf_hbm, pf_hbm, ub_hbm, pb_hbm, g_hbm,
                  out_hbm, uidx_v, pidx_v, urow_v, prow_v, ub_v, pb_v, g_v,
                  out_v, sem_u, sem_p, sem_ub, sem_pb):
        wid = lax.axis_index("s") * nc + lax.axis_index("c")
        base = wid * b_per_w

        pltpu.sync_copy(uidx_hbm.at[pl.ds(base, b_per_w)], uidx_v)
        pltpu.sync_copy(pidx_hbm.at[pl.ds(base, b_per_w)], pidx_v)
        pltpu.sync_copy(g_hbm, g_v)

        cp_u = pltpu.async_copy(uf_hbm.at[uidx_v], urow_v, sem_u)
        cp_p = pltpu.async_copy(pf_hbm.at[pidx_v], prow_v, sem_p)
        cp_ub = pltpu.async_copy(ub_hbm.at[uidx_v], ub_v, sem_ub)
        cp_pb = pltpu.async_copy(pb_hbm.at[pidx_v], pb_v, sem_pb)
        cp_u.wait()
        cp_p.wait()
        cp_ub.wait()
        cp_pb.wait()

        lanes = lax.iota(jnp.int32, _L)
        gvec = g_v[...]

        def group_body(g, _):
            off = pl.multiple_of(g * _L, _L)
            rows = off + lanes
            acc = gvec
            for d in range(_NUM_FACTORS):
                cols = jnp.full((_L,), d, jnp.int32)
                u = plsc.load_gather(urow_v, [rows, cols])
                p = plsc.load_gather(prow_v, [rows, cols])
                acc = acc + u * p
            out_v[pl.ds(off, _L)] = acc + ub_v[pl.ds(off, _L)] + pb_v[pl.ds(off, _L)]
            return _

        lax.fori_loop(0, n_groups, group_body, None)
        pltpu.sync_copy(out_v, out_hbm.at[pl.ds(base, b_per_w)])

    return mf_kernel


def kernel(user_indices, post_indices, user_factors, post_factors,
           user_intercepts, post_intercepts, global_intercept):
    info = plsc.get_sparse_core_info()
    nc, ns = info.num_cores, info.num_subcores
    batch = user_indices.shape[0]
    call = _build_call(batch, nc * ns, nc)
    return call(
        user_indices.astype(jnp.int32),
        post_indices.astype(jnp.int32),
        user_factors,
        post_factors,
        user_intercepts.reshape(-1),
        post_intercepts.reshape(-1),
        jnp.broadcast_to(global_intercept.astype(jnp.float32), (_L,)),
    )
